# baseline (speedup 1.0000x reference)
import functools

import jax
import jax.numpy as jnp
from jax import lax
from jax.experimental import pallas as pl
from jax.experimental.pallas import tpu as pltpu
from jax.experimental.pallas import tpu_sc as plsc

B = 4096; L = 200; HIST = 50; ALEN = 20
V = 100001; D_ID = 64; D_SP = 32; H = 64; DNN = 128
LH = 104
LP = 2 * LH
VROWS = 1024
VP = ((V + VROWS - 1) // VROWS) * VROWS
BBLK = 512


def _pmat_body(tab, w1, b1, out):
    out[...] = jnp.dot(tab[...], w1[...],
                       preferred_element_type=jnp.float32) + b1[...]


def _precompute_p(tab_pad, item_w1, item_b1):
    return pl.pallas_call(
        _pmat_body,
        grid=(VP // VROWS,),
        in_specs=[
            pl.BlockSpec((VROWS, D_ID), lambda i: (i, 0)),
            pl.BlockSpec((D_ID, DNN), lambda i: (0, 0)),
            pl.BlockSpec((1, DNN), lambda i: (0, 0)),
        ],
        out_specs=pl.BlockSpec((VROWS, DNN), lambda i: (i, 0)),
        out_shape=jax.ShapeDtypeStruct((VP, DNN), jnp.float32),
    )(tab_pad, item_w1, item_b1.reshape(1, DNN))


def _make_sc_pool(nc, ns):
    nw = nc * ns
    bpw = B // nw
    nck = bpw // 4
    mesh = plsc.VectorSubcoreMesh(core_axis_name="c", subcore_axis_name="s")
    PNB = 2

    @functools.partial(
        pl.kernel,
        out_type=(
            jax.ShapeDtypeStruct((B, D_ID), jnp.float32),
            jax.ShapeDtypeStruct((B, D_SP), jnp.float32),
            jax.ShapeDtypeStruct((B, D_ID), jnp.float32),
            jax.ShapeDtypeStruct((B, D_SP), jnp.float32),
            jax.ShapeDtypeStruct((B, D_SP), jnp.float32),
            jax.ShapeDtypeStruct((B, D_SP), jnp.float32),
            jax.ShapeDtypeStruct((B, D_SP), jnp.float32),
        ),
        mesh=mesh,
        scratch_types=(
            pltpu.VMEM((nck, 2, 128), jnp.int32),
            pltpu.VMEM((nck, 2, 64), jnp.int32),
            pltpu.VMEM((5, bpw), jnp.int32),
            tuple(pltpu.VMEM((256, D_ID), jnp.float32) for _ in range(PNB)),
            tuple(pltpu.VMEM((128, D_SP), jnp.float32) for _ in range(PNB)),
            pltpu.VMEM((bpw, D_ID), jnp.float32),
            pltpu.VMEM((bpw, D_SP), jnp.float32),
            pltpu.VMEM((bpw, D_SP), jnp.float32),
            pltpu.VMEM((bpw, D_SP), jnp.float32),
            pltpu.VMEM((bpw, D_SP), jnp.float32),
            pltpu.VMEM((bpw, D_ID), jnp.float32),
            pltpu.VMEM((bpw, D_SP), jnp.float32),
            tuple(pltpu.SemaphoreType.DMA for _ in range(PNB)),
            pltpu.SemaphoreType.DMA,
            pltpu.SemaphoreType.DMA,
        ),
        compiler_params=pltpu.CompilerParams(use_tc_tiling_on_sc=False),
    )
    def sc_pool(item_t, user_t, sp1_t, sp2_t, arr_t, csp1_t, csp2_t,
                histp, uarrp, uid, usp1, usp2, csp1, csp2,
                hist_sum, uarr_sum, user_row, usp1_row, usp2_row,
                csp1_row, csp2_row,
                hidx, aidx, sidx, hb, ab, rb64, rb32a, rb32b, rb32c, rb32d,
                hs, us, gs, ssem, wsem):
        wid = lax.axis_index("s") * nc + lax.axis_index("c")
        bs = wid * bpw
        pltpu.sync_copy(histp.at[pl.ds(wid * nck, nck)], hidx)
        pltpu.sync_copy(uarrp.at[pl.ds(wid * nck, nck)], aidx)
        for k, src in enumerate((uid, usp1, usp2, csp1, csp2)):
            pltpu.sync_copy(src.at[pl.ds(bs, bpw)], sidx.at[k])

        sing = ((user_t, user_row, rb64), (sp1_t, usp1_row, rb32a),
                (sp2_t, usp2_row, rb32b), (csp1_t, csp1_row, rb32c),
                (csp2_t, csp2_row, rb32d))
        for k, (tab, out, buf) in enumerate(sing):
            pltpu.async_copy(tab.at[sidx.at[k]], buf, ssem)

        def _start(d, u):
            pltpu.async_copy(item_t.at[hidx.at[d, 0]],
                             hb[u].at[pl.ds(0, 128)], gs[u])
            pltpu.async_copy(item_t.at[hidx.at[d, 1]],
                             hb[u].at[pl.ds(128, 128)], gs[u])
            pltpu.async_copy(arr_t.at[aidx.at[d, 0]],
                             ab[u].at[pl.ds(0, 64)], gs[u])
            pltpu.async_copy(arr_t.at[aidx.at[d, 1]],
                             ab[u].at[pl.ds(64, 64)], gs[u])

        def _wait(u):
            for buf in (hb[u].at[pl.ds(0, 128)], hb[u].at[pl.ds(128, 128)]):
                pltpu.make_async_copy(item_t.at[hidx.at[0, 0]], buf,
                                      gs[u]).wait()
            for buf in (ab[u].at[pl.ds(0, 64)], ab[u].at[pl.ds(64, 64)]):
                pltpu.make_async_copy(arr_t.at[aidx.at[0, 0]], buf,
                                      gs[u]).wait()

        def _pool(d, u):
            for half in range(4):
                b = 4 * d + half

                def hsum(t, acc):
                    for i in range(16):
                        row = half * 64 + t * 16 + i
                        acc = tuple(
                            acc[j] + hb[u][row, pl.ds(j * 16, 16)]
                            for j in range(4))
                    return acc

                acc = lax.fori_loop(
                    0, 4, hsum,
                    tuple(jnp.zeros((16,), jnp.float32) for _ in range(4)))
                for j in range(4):
                    hs[b, pl.ds(j * 16, 16)] = acc[j]

                def asum(t, acc):
                    for i in range(16):
                        row = half * 32 + t * 16 + i
                        acc = tuple(
                            acc[j] + ab[u][row, pl.ds(j * 16, 16)]
                            for j in range(2))
                    return acc

                acc2 = lax.fori_loop(
                    0, 2, asum,
                    tuple(jnp.zeros((16,), jnp.float32) for _ in range(2)))
                for j in range(2):
                    us[b, pl.ds(j * 16, 16)] = acc2[j]

        for u in range(PNB - 1):
            _start(u, u)

        def body(k, carry):
            d0 = PNB * k
            for u in range(PNB):
                d = d0 + u
                pl.when(d + PNB - 1 < nck)(
                    functools.partial(_start, d + PNB - 1, (u + PNB - 1) % PNB))
                _wait(u)
                _pool(d, u)
            return carry

        lax.fori_loop(0, nck // PNB, body, 0)

        pltpu.async_copy(hs, hist_sum.at[pl.ds(bs, bpw)], wsem)
        pltpu.async_copy(us, uarr_sum.at[pl.ds(bs, bpw)], wsem)
        for k, (tab, out, buf) in enumerate(sing):
            pltpu.make_async_copy(tab.at[sidx.at[k]], buf, ssem).wait()
            pltpu.async_copy(buf, out.at[pl.ds(bs, bpw)], wsem)
        pltpu.make_async_copy(hs, hist_sum.at[pl.ds(bs, bpw)], wsem).wait()
        pltpu.make_async_copy(us, uarr_sum.at[pl.ds(bs, bpw)], wsem).wait()
        for _, out, buf in sing:
            pltpu.make_async_copy(buf, out.at[pl.ds(bs, bpw)], wsem).wait()

    return sc_pool


def _towers_body(hist_sum, histp, uarr_sum, user_row, usp1, usp2, csp1, csp2,
                 uw1, ub1, uw2, ub2, cw1, cb1, cw2, cb2, dw, db,
                 iw2, ib2, mw, mb, v_out, cb_out):
    f32 = jnp.float32
    dot = functools.partial(jnp.dot, preferred_element_type=f32)
    cnt = jnp.sum((histp[...] != 0).astype(f32), axis=1, keepdims=True)
    seq_emb = hist_sum[...] / jnp.maximum(cnt, 1.0)
    uh = (dot(user_row[...], uw1[0:64]) + dot(usp1[...], uw1[64:96])
          + dot(usp2[...], uw1[96:128]) + dot(uarr_sum[...], uw1[128:160])
          + ub1[...])
    user_h = dot(jnp.maximum(uh, 0.0), uw2[...]) + ub2[...]
    ch = (dot(csp1[...], cw1[0:32]) + dot(csp2[...], cw1[32:64])
          + dot(seq_emb, cw1[64:128]) + cb1[...])
    ctx_h = dot(jnp.maximum(ch, 0.0), cw2[...]) + cb2[...]
    query = dot(user_h, dw[0:64]) + dot(ctx_h, dw[64:128]) + db[...]
    amat = dot(iw2[...], mw[...])
    v_out[...] = lax.dot_general(query, amat, (((1,), (1,)), ((), ())),
                                 preferred_element_type=f32)
    w0 = dot(ib2[...], mw[...]) + mb[...]
    cbv = lax.dot_general(query, w0, (((1,), (1,)), ((), ())),
                          preferred_element_type=f32)
    cb_out[...] = jnp.broadcast_to(cbv, (cbv.shape[0], DNN))


def _towers(hist_sum, histp, uarr_sum, user_row, usp1, usp2, csp1, csp2,
            uw1, ub1, uw2, ub2, cw1, cb1, cw2, cb2, dw, db, iw2, ib2, mw, mb):
    rep = lambda shape: pl.BlockSpec(shape, lambda i: tuple(0 for _ in shape))
    blk = lambda width: pl.BlockSpec((BBLK, width), lambda i: (i, 0))
    return pl.pallas_call(
        _towers_body,
        grid=(B // BBLK,),
        in_specs=[
            blk(D_ID), blk(64), blk(D_SP), blk(D_ID), blk(D_SP), blk(D_SP),
            blk(D_SP), blk(D_SP),
            rep((D_ID + 3 * D_SP, DNN)), rep((1, DNN)), rep((DNN, H)),
            rep((1, H)),
            rep((2 * D_SP + D_ID, DNN)), rep((1, DNN)), rep((DNN, H)),
            rep((1, H)),
            rep((2 * H, H)), rep((1, H)),
            rep((DNN, H)), rep((1, H)), rep((H, H)), rep((1, H)),
        ],
        out_specs=(blk(DNN), blk(DNN)),
        out_shape=(jax.ShapeDtypeStruct((B, DNN), jnp.float32),
                   jax.ShapeDtypeStruct((B, DNN), jnp.float32)),
    )(hist_sum, histp, uarr_sum, user_row, usp1, usp2, csp1, csp2,
      uw1, ub1.reshape(1, DNN), uw2, ub2.reshape(1, H),
      cw1, cb1.reshape(1, DNN), cw2, cb2.reshape(1, H),
      dw, db.reshape(1, H), iw2, ib2.reshape(1, H), mw, mb.reshape(1, H))


def _make_sc_score(nc, ns):
    nw = nc * ns
    bpw = B // nw
    mesh = plsc.VectorSubcoreMesh(core_axis_name="c", subcore_axis_name="s")

    @functools.partial(
        pl.kernel,
        out_type=jax.ShapeDtypeStruct((B, LP), jnp.float32),
        mesh=mesh,
        scratch_types=(
            pltpu.VMEM((bpw, 2, LH), jnp.int32),
            pltpu.VMEM((bpw, DNN), jnp.float32),
            pltpu.VMEM((bpw, 16), jnp.float32),
            pltpu.VMEM((LP, DNN), jnp.float32),
            pltpu.VMEM((LP, DNN), jnp.float32),
            pltpu.VMEM((16, 16), jnp.float32),
            pltpu.VMEM((LP,), jnp.float32),
            pltpu.VMEM((LP,), jnp.float32),
            pltpu.SemaphoreType.DMA,
            pltpu.SemaphoreType.DMA,
            pltpu.SemaphoreType.DMA,
            pltpu.SemaphoreType.DMA,
        ),
        compiler_params=pltpu.CompilerParams(use_tc_tiling_on_sc=False,
                                             needs_layout_passes=False),
    )
    def sc_score(p_hbm, idx_hbm, v_hbm, cb_hbm, out_hbm,
                 idxv, vv, cbv, z0, z1, m, s0, s1, g0, g1, ws0, ws1):
        wid = lax.axis_index("s") * nc + lax.axis_index("c")
        bs = wid * bpw
        pltpu.sync_copy(idx_hbm.at[pl.ds(bs, bpw)], idxv)
        pltpu.sync_copy(v_hbm.at[pl.ds(bs, bpw)], vv)
        pltpu.sync_copy(cb_hbm.at[pl.ds(bs, bpw), pl.ds(0, 16)], cbv)
        iota16 = lax.iota(jnp.int32, 16)

        def _gather(b, z, g):
            pltpu.async_copy(p_hbm.at[idxv.at[b, 0]], z.at[pl.ds(0, LH)], g)
            pltpu.async_copy(p_hbm.at[idxv.at[b, 1]], z.at[pl.ds(LH, LH)], g)

        def _gwait(b, z, g):
            pltpu.make_async_copy(p_hbm.at[idxv.at[b, 0]],
                                  z.at[pl.ds(0, LH)], g).wait()
            pltpu.make_async_copy(p_hbm.at[idxv.at[b, 1]],
                                  z.at[pl.ds(LH, LH)], g).wait()

        def _compute(b, z, s):
            vb = [vv[b, pl.ds(j * 16, 16)] for j in range(8)]
            cb_vec = cbv[b, :]

            def grp(g_, carry):
                r = g_ * 16
                for i in range(16):
                    pr = [jnp.maximum(z[r + i, pl.ds(j * 16, 16)], 0.0) * vb[j]
                          for j in range(8)]
                    q = [pr[0] + pr[1], pr[2] + pr[3],
                         pr[4] + pr[5], pr[6] + pr[7]]
                    m[i, :] = (q[0] + q[1]) + (q[2] + q[3])
                sc = cb_vec
                for k in range(16):
                    sc = sc + plsc.load_gather(
                        m, [iota16, jnp.full((16,), k, jnp.int32)])
                s[pl.ds(r, 16)] = sc
                return carry

            lax.fori_loop(0, LP // 16, grp, 0)

        def _swrite(b, s, ws):
            pltpu.async_copy(s, out_hbm.at[bs + b], ws)

        def _sdrain(s, ws):
            pltpu.make_async_copy(s, out_hbm.at[0], ws).wait()

        _gather(0, z0, g0)

        def body(k, carry):
            c = 2 * k
            pl.when(c + 1 < bpw)(lambda: _gather(c + 1, z1, g1))
            _gwait(c, z0, g0)
            pl.when(c >= 2)(lambda: _sdrain(s0, ws0))
            _compute(c, z0, s0)
            _swrite(c, s0, ws0)
            pl.when(c + 2 < bpw)(lambda: _gather(c + 2, z0, g0))
            _gwait(c + 1, z1, g1)
            pl.when(c >= 2)(lambda: _sdrain(s1, ws1))
            _compute(c + 1, z1, s1)
            _swrite(c + 1, s1, ws1)
            return carry

        lax.fori_loop(0, bpw // 2, body, 0)
        _sdrain(s0, ws0)
        _sdrain(s1, ws1)

    return sc_score


def kernel(item_table, user_table, u_sp1_t, u_sp2_t, u_arr_t, c_sp1_t, c_sp2_t,
           item_W1, item_b1, item_W2, item_b2,
           user_W1, user_b1, user_W2, user_b2,
           ctx_W1, ctx_b1, ctx_W2, ctx_b2,
           merge_W, merge_b, ctxdnn_W, ctxdnn_b,
           seq_id, item_mask, user_id, u_sp1, u_sp2, u_arr, c_sp1, c_sp2,
           hist):
    try:
        info = plsc.get_sparse_core_info()
        nc, ns = info.num_cores, info.num_subcores
    except Exception:
        nc, ns = 2, 16

    tab_pad = jnp.pad(item_table, ((0, VP - V), (0, 0)))
    p_full = _precompute_p(tab_pad, item_W1, item_b1)

    histp = jnp.pad(hist, ((0, 0), (0, 64 - HIST)))
    uarrp = jnp.pad(u_arr, ((0, 0), (0, 32 - ALEN)))
    (hist_sum, uarr_sum, user_row, usp1_row, usp2_row, csp1_row,
     csp2_row) = _make_sc_pool(nc, ns)(
        item_table, user_table, u_sp1_t, u_sp2_t, u_arr_t, c_sp1_t, c_sp2_t,
        histp.reshape(B // 4, 2, 128), uarrp.reshape(B // 4, 2, 64),
        user_id, u_sp1, u_sp2, c_sp1, c_sp2)

    v, cb = _towers(hist_sum, histp, uarr_sum, user_row, usp1_row, usp2_row,
                    csp1_row, csp2_row,
                    user_W1, user_b1, user_W2, user_b2,
                    ctx_W1, ctx_b1, ctx_W2, ctx_b2,
                    ctxdnn_W, ctxdnn_b, item_W2, item_b2, merge_W, merge_b)

    idx = (seq_id * item_mask).astype(jnp.int32)
    idx3 = jnp.pad(idx, ((0, 0), (0, LP - L))).reshape(B, 2, LH)
    scores_pad = _make_sc_score(nc, ns)(p_full, idx3, v, cb)
    return scores_pad[:, :L]


# speedup vs baseline: 1.2858x; 1.2858x over previous
import functools

import jax
import jax.numpy as jnp
from jax import lax
from jax.experimental import pallas as pl
from jax.experimental.pallas import tpu as pltpu
from jax.experimental.pallas import tpu_sc as plsc

B = 4096; L = 200; HIST = 50; ALEN = 20
V = 100001; D_ID = 64; D_SP = 32; H = 64; DNN = 128
LH = 104
LP = 2 * LH
VROWS = 1024
VP = ((V + VROWS - 1) // VROWS) * VROWS
BBLK = 512


def _pmat_body(tab, w1, b1, out):
    out[...] = (jnp.dot(tab[...], w1[...],
                        preferred_element_type=jnp.float32)
                + b1[...]).astype(jnp.bfloat16)


def _precompute_p(tab_pad, item_w1, item_b1):
    return pl.pallas_call(
        _pmat_body,
        grid=(VP // VROWS,),
        in_specs=[
            pl.BlockSpec((VROWS, D_ID), lambda i: (i, 0)),
            pl.BlockSpec((D_ID, DNN), lambda i: (0, 0)),
            pl.BlockSpec((1, DNN), lambda i: (0, 0)),
        ],
        out_specs=pl.BlockSpec((VROWS, DNN), lambda i: (i, 0)),
        out_shape=jax.ShapeDtypeStruct((VP, DNN), jnp.bfloat16),
    )(tab_pad, item_w1, item_b1.reshape(1, DNN))


def _make_sc_pool(nc, ns):
    nw = nc * ns
    bpw = B // nw
    nck = bpw // 4
    mesh = plsc.VectorSubcoreMesh(core_axis_name="c", subcore_axis_name="s")
    PNB = 2

    @functools.partial(
        pl.kernel,
        out_type=(
            jax.ShapeDtypeStruct((B, D_ID), jnp.float32),
            jax.ShapeDtypeStruct((B, D_SP), jnp.float32),
            jax.ShapeDtypeStruct((B, D_ID), jnp.float32),
            jax.ShapeDtypeStruct((B, D_SP), jnp.float32),
            jax.ShapeDtypeStruct((B, D_SP), jnp.float32),
            jax.ShapeDtypeStruct((B, D_SP), jnp.float32),
            jax.ShapeDtypeStruct((B, D_SP), jnp.float32),
        ),
        mesh=mesh,
        scratch_types=(
            pltpu.VMEM((nck, 2, 128), jnp.int32),
            pltpu.VMEM((nck, 2, 64), jnp.int32),
            pltpu.VMEM((5, bpw), jnp.int32),
            tuple(pltpu.VMEM((256, D_ID), jnp.float32) for _ in range(PNB)),
            tuple(pltpu.VMEM((128, D_SP), jnp.float32) for _ in range(PNB)),
            pltpu.VMEM((bpw, D_ID), jnp.float32),
            pltpu.VMEM((bpw, D_SP), jnp.float32),
            pltpu.VMEM((bpw, D_SP), jnp.float32),
            pltpu.VMEM((bpw, D_SP), jnp.float32),
            pltpu.VMEM((bpw, D_SP), jnp.float32),
            pltpu.VMEM((bpw, D_ID), jnp.float32),
            pltpu.VMEM((bpw, D_SP), jnp.float32),
            tuple(pltpu.SemaphoreType.DMA for _ in range(PNB)),
            pltpu.SemaphoreType.DMA,
            pltpu.SemaphoreType.DMA,
        ),
        compiler_params=pltpu.CompilerParams(use_tc_tiling_on_sc=False),
    )
    def sc_pool(item_t, user_t, sp1_t, sp2_t, arr_t, csp1_t, csp2_t,
                histp, uarrp, uid, usp1, usp2, csp1, csp2,
                hist_sum, uarr_sum, user_row, usp1_row, usp2_row,
                csp1_row, csp2_row,
                hidx, aidx, sidx, hb, ab, rb64, rb32a, rb32b, rb32c, rb32d,
                hs, us, gs, ssem, wsem):
        wid = lax.axis_index("s") * nc + lax.axis_index("c")
        bs = wid * bpw
        pltpu.sync_copy(histp.at[pl.ds(wid * nck, nck)], hidx)
        pltpu.sync_copy(uarrp.at[pl.ds(wid * nck, nck)], aidx)
        for k, src in enumerate((uid, usp1, usp2, csp1, csp2)):
            pltpu.sync_copy(src.at[pl.ds(bs, bpw)], sidx.at[k])

        sing = ((user_t, user_row, rb64), (sp1_t, usp1_row, rb32a),
                (sp2_t, usp2_row, rb32b), (csp1_t, csp1_row, rb32c),
                (csp2_t, csp2_row, rb32d))
        for k, (tab, out, buf) in enumerate(sing):
            pltpu.async_copy(tab.at[sidx.at[k]], buf, ssem)

        def _start(d, u):
            pltpu.async_copy(item_t.at[hidx.at[d, 0]],
                             hb[u].at[pl.ds(0, 128)], gs[u])
            pltpu.async_copy(item_t.at[hidx.at[d, 1]],
                             hb[u].at[pl.ds(128, 128)], gs[u])
            pltpu.async_copy(arr_t.at[aidx.at[d, 0]],
                             ab[u].at[pl.ds(0, 64)], gs[u])
            pltpu.async_copy(arr_t.at[aidx.at[d, 1]],
                             ab[u].at[pl.ds(64, 64)], gs[u])

        def _wait(u):
            for buf in (hb[u].at[pl.ds(0, 128)], hb[u].at[pl.ds(128, 128)]):
                pltpu.make_async_copy(item_t.at[hidx.at[0, 0]], buf,
                                      gs[u]).wait()
            for buf in (ab[u].at[pl.ds(0, 64)], ab[u].at[pl.ds(64, 64)]):
                pltpu.make_async_copy(arr_t.at[aidx.at[0, 0]], buf,
                                      gs[u]).wait()

        def _pool(d, u):
            for half in range(4):
                b = 4 * d + half

                def hsum(t, acc):
                    for i in range(16):
                        row = half * 64 + t * 16 + i
                        acc = tuple(
                            acc[j] + hb[u][row, pl.ds(j * 16, 16)]
                            for j in range(4))
                    return acc

                acc = lax.fori_loop(
                    0, 4, hsum,
                    tuple(jnp.zeros((16,), jnp.float32) for _ in range(4)))
                for j in range(4):
                    hs[b, pl.ds(j * 16, 16)] = acc[j]

                def asum(t, acc):
                    for i in range(16):
                        row = half * 32 + t * 16 + i
                        acc = tuple(
                            acc[j] + ab[u][row, pl.ds(j * 16, 16)]
                            for j in range(2))
                    return acc

                acc2 = lax.fori_loop(
                    0, 2, asum,
                    tuple(jnp.zeros((16,), jnp.float32) for _ in range(2)))
                for j in range(2):
                    us[b, pl.ds(j * 16, 16)] = acc2[j]

        for u in range(PNB - 1):
            _start(u, u)

        def body(k, carry):
            d0 = PNB * k
            for u in range(PNB):
                d = d0 + u
                pl.when(d + PNB - 1 < nck)(
                    functools.partial(_start, d + PNB - 1, (u + PNB - 1) % PNB))
                _wait(u)
                _pool(d, u)
            return carry

        lax.fori_loop(0, nck // PNB, body, 0)

        pltpu.async_copy(hs, hist_sum.at[pl.ds(bs, bpw)], wsem)
        pltpu.async_copy(us, uarr_sum.at[pl.ds(bs, bpw)], wsem)
        for k, (tab, out, buf) in enumerate(sing):
            pltpu.make_async_copy(tab.at[sidx.at[k]], buf, ssem).wait()
            pltpu.async_copy(buf, out.at[pl.ds(bs, bpw)], wsem)
        pltpu.make_async_copy(hs, hist_sum.at[pl.ds(bs, bpw)], wsem).wait()
        pltpu.make_async_copy(us, uarr_sum.at[pl.ds(bs, bpw)], wsem).wait()
        for _, out, buf in sing:
            pltpu.make_async_copy(buf, out.at[pl.ds(bs, bpw)], wsem).wait()

    return sc_pool


def _towers_body(hist_sum, histp, uarr_sum, user_row, usp1, usp2, csp1, csp2,
                 uw1, ub1, uw2, ub2, cw1, cb1, cw2, cb2, dw, db,
                 iw2, ib2, mw, mb, v_out, cb_out):
    f32 = jnp.float32
    dot = functools.partial(jnp.dot, preferred_element_type=f32)
    cnt = jnp.sum((histp[...] != 0).astype(f32), axis=1, keepdims=True)
    seq_emb = hist_sum[...] / jnp.maximum(cnt, 1.0)
    uh = (dot(user_row[...], uw1[0:64]) + dot(usp1[...], uw1[64:96])
          + dot(usp2[...], uw1[96:128]) + dot(uarr_sum[...], uw1[128:160])
          + ub1[...])
    user_h = dot(jnp.maximum(uh, 0.0), uw2[...]) + ub2[...]
    ch = (dot(csp1[...], cw1[0:32]) + dot(csp2[...], cw1[32:64])
          + dot(seq_emb, cw1[64:128]) + cb1[...])
    ctx_h = dot(jnp.maximum(ch, 0.0), cw2[...]) + cb2[...]
    query = dot(user_h, dw[0:64]) + dot(ctx_h, dw[64:128]) + db[...]
    amat = dot(iw2[...], mw[...])
    v_out[...] = lax.dot_general(query, amat, (((1,), (1,)), ((), ())),
                                 preferred_element_type=f32)
    w0 = dot(ib2[...], mw[...]) + mb[...]
    cbv = lax.dot_general(query, w0, (((1,), (1,)), ((), ())),
                          preferred_element_type=f32)
    cb_out[...] = jnp.broadcast_to(cbv, (cbv.shape[0], DNN))


def _towers(hist_sum, histp, uarr_sum, user_row, usp1, usp2, csp1, csp2,
            uw1, ub1, uw2, ub2, cw1, cb1, cw2, cb2, dw, db, iw2, ib2, mw, mb):
    rep = lambda shape: pl.BlockSpec(shape, lambda i: tuple(0 for _ in shape))
    blk = lambda width: pl.BlockSpec((BBLK, width), lambda i: (i, 0))
    return pl.pallas_call(
        _towers_body,
        grid=(B // BBLK,),
        in_specs=[
            blk(D_ID), blk(64), blk(D_SP), blk(D_ID), blk(D_SP), blk(D_SP),
            blk(D_SP), blk(D_SP),
            rep((D_ID + 3 * D_SP, DNN)), rep((1, DNN)), rep((DNN, H)),
            rep((1, H)),
            rep((2 * D_SP + D_ID, DNN)), rep((1, DNN)), rep((DNN, H)),
            rep((1, H)),
            rep((2 * H, H)), rep((1, H)),
            rep((DNN, H)), rep((1, H)), rep((H, H)), rep((1, H)),
        ],
        out_specs=(blk(DNN), blk(DNN)),
        out_shape=(jax.ShapeDtypeStruct((B, DNN), jnp.float32),
                   jax.ShapeDtypeStruct((B, DNN), jnp.float32)),
    )(hist_sum, histp, uarr_sum, user_row, usp1, usp2, csp1, csp2,
      uw1, ub1.reshape(1, DNN), uw2, ub2.reshape(1, H),
      cw1, cb1.reshape(1, DNN), cw2, cb2.reshape(1, H),
      dw, db.reshape(1, H), iw2, ib2.reshape(1, H), mw, mb.reshape(1, H))


def _make_sc_score(nc, ns):
    nw = nc * ns
    bpw = B // nw
    mesh = plsc.VectorSubcoreMesh(core_axis_name="c", subcore_axis_name="s")

    @functools.partial(
        pl.kernel,
        out_type=jax.ShapeDtypeStruct((B, LP), jnp.float32),
        mesh=mesh,
        scratch_types=(
            pltpu.VMEM((bpw, 2, LH), jnp.int32),
            pltpu.VMEM((bpw, DNN), jnp.float32),
            pltpu.VMEM((bpw, 16), jnp.float32),
            pltpu.VMEM((LP, DNN), jnp.bfloat16),
            pltpu.VMEM((LP, DNN), jnp.bfloat16),
            pltpu.VMEM((16, 16), jnp.float32),
            pltpu.VMEM((LP,), jnp.float32),
            pltpu.VMEM((LP,), jnp.float32),
            pltpu.SemaphoreType.DMA,
            pltpu.SemaphoreType.DMA,
            pltpu.SemaphoreType.DMA,
            pltpu.SemaphoreType.DMA,
        ),
        compiler_params=pltpu.CompilerParams(use_tc_tiling_on_sc=False,
                                             needs_layout_passes=False),
    )
    def sc_score(p_hbm, idx_hbm, v_hbm, cb_hbm, out_hbm,
                 idxv, vv, cbv, z0, z1, m, s0, s1, g0, g1, ws0, ws1):
        wid = lax.axis_index("s") * nc + lax.axis_index("c")
        bs = wid * bpw
        pltpu.sync_copy(idx_hbm.at[pl.ds(bs, bpw)], idxv)
        pltpu.sync_copy(v_hbm.at[pl.ds(bs, bpw)], vv)
        pltpu.sync_copy(cb_hbm.at[pl.ds(bs, bpw), pl.ds(0, 16)], cbv)
        iota16 = lax.iota(jnp.int32, 16)

        def _gather(b, z, g):
            pltpu.async_copy(p_hbm.at[idxv.at[b, 0]], z.at[pl.ds(0, LH)], g)
            pltpu.async_copy(p_hbm.at[idxv.at[b, 1]], z.at[pl.ds(LH, LH)], g)

        def _gwait(b, z, g):
            pltpu.make_async_copy(p_hbm.at[idxv.at[b, 0]],
                                  z.at[pl.ds(0, LH)], g).wait()
            pltpu.make_async_copy(p_hbm.at[idxv.at[b, 1]],
                                  z.at[pl.ds(LH, LH)], g).wait()

        def _compute(b, z, s):
            vb = [vv[b, pl.ds(j * 16, 16)] for j in range(8)]
            cb_vec = cbv[b, :]

            def grp(g_, carry):
                r = g_ * 16
                for i in range(16):
                    pr = []
                    for j in range(4):
                        zr = jnp.maximum(z[r + i, pl.ds(j * 32, 32)],
                                         jnp.bfloat16(0.0))
                        za, zc = plsc.unpack(zr,
                                             format=plsc.PackFormat.INTERLEAVED)
                        pr.append(za * vb[2 * j])
                        pr.append(zc * vb[2 * j + 1])
                    q = [pr[0] + pr[1], pr[2] + pr[3],
                         pr[4] + pr[5], pr[6] + pr[7]]
                    m[i, :] = (q[0] + q[1]) + (q[2] + q[3])
                sc = cb_vec
                for k in range(16):
                    sc = sc + plsc.load_gather(
                        m, [iota16, jnp.full((16,), k, jnp.int32)])
                s[pl.ds(r, 16)] = sc
                return carry

            lax.fori_loop(0, LP // 16, grp, 0)

        def _swrite(b, s, ws):
            pltpu.async_copy(s, out_hbm.at[bs + b], ws)

        def _sdrain(s, ws):
            pltpu.make_async_copy(s, out_hbm.at[0], ws).wait()

        _gather(0, z0, g0)

        def body(k, carry):
            c = 2 * k
            pl.when(c + 1 < bpw)(lambda: _gather(c + 1, z1, g1))
            _gwait(c, z0, g0)
            pl.when(c >= 2)(lambda: _sdrain(s0, ws0))
            _compute(c, z0, s0)
            _swrite(c, s0, ws0)
            pl.when(c + 2 < bpw)(lambda: _gather(c + 2, z0, g0))
            _gwait(c + 1, z1, g1)
            pl.when(c >= 2)(lambda: _sdrain(s1, ws1))
            _compute(c + 1, z1, s1)
            _swrite(c + 1, s1, ws1)
            return carry

        lax.fori_loop(0, bpw // 2, body, 0)
        _sdrain(s0, ws0)
        _sdrain(s1, ws1)

    return sc_score


def kernel(item_table, user_table, u_sp1_t, u_sp2_t, u_arr_t, c_sp1_t, c_sp2_t,
           item_W1, item_b1, item_W2, item_b2,
           user_W1, user_b1, user_W2, user_b2,
           ctx_W1, ctx_b1, ctx_W2, ctx_b2,
           merge_W, merge_b, ctxdnn_W, ctxdnn_b,
           seq_id, item_mask, user_id, u_sp1, u_sp2, u_arr, c_sp1, c_sp2,
           hist):
    try:
        info = plsc.get_sparse_core_info()
        nc, ns = info.num_cores, info.num_subcores
    except Exception:
        nc, ns = 2, 16

    tab_pad = jnp.pad(item_table, ((0, VP - V), (0, 0)))
    p_full = _precompute_p(tab_pad, item_W1, item_b1)

    histp = jnp.pad(hist, ((0, 0), (0, 64 - HIST)))
    uarrp = jnp.pad(u_arr, ((0, 0), (0, 32 - ALEN)))
    (hist_sum, uarr_sum, user_row, usp1_row, usp2_row, csp1_row,
     csp2_row) = _make_sc_pool(nc, ns)(
        item_table, user_table, u_sp1_t, u_sp2_t, u_arr_t, c_sp1_t, c_sp2_t,
        histp.reshape(B // 4, 2, 128), uarrp.reshape(B // 4, 2, 64),
        user_id, u_sp1, u_sp2, c_sp1, c_sp2)

    perm = jnp.array(sum([[32 * j + 2 * i for i in range(16)]
                          + [32 * j + 2 * i + 1 for i in range(16)]
                          for j in range(4)], []), dtype=jnp.int32)
    v, cb = _towers(hist_sum, histp, uarr_sum, user_row, usp1_row, usp2_row,
                    csp1_row, csp2_row,
                    user_W1, user_b1, user_W2, user_b2,
                    ctx_W1, ctx_b1, ctx_W2, ctx_b2,
                    ctxdnn_W, ctxdnn_b, item_W2[perm], item_b2, merge_W,
                    merge_b)

    idx = (seq_id * item_mask).astype(jnp.int32)
    idx3 = jnp.pad(idx, ((0, 0), (0, LP - L))).reshape(B, 2, LH)
    scores_pad = _make_sc_score(nc, ns)(p_full, idx3, v, cb)
    return scores_pad[:, :L]
